# Initial kernel scaffold; baseline (speedup 1.0000x reference)
import jax
import jax.numpy as jnp
from jax.experimental import pallas as pl


def kernel(x, edge_index, W_fc, b_fc, W_root1, W_nbr1, b1, W_root2, W_nbr2, b2, W_logit, b_logit, W_val, b_val):
    raise NotImplementedError("write your pallas kernel here")


# speedup vs baseline: 3.0766x; 3.0766x over previous
import functools

import jax
import jax.numpy as jnp
from jax import lax
from jax.experimental import pallas as pl
from jax.experimental.pallas import tpu as pltpu
from jax.experimental.pallas import tpu_sc as plsc

N = 10000
E = 320000
D_IN = 128
D_H = 64
D_OUT = 19

N_PAD = 10112
NC, NS = 2, 16
NW = NC * NS
CH = 128
CPW = -(-E // (NW * CH))
E_PAD = NW * CPW * CH
RPT = N_PAD // NS
GW = 128


def _mask_rows(a):
    row = lax.broadcasted_iota(jnp.int32, a.shape, 0)
    return jnp.where(row < N, a, 0.0)


def _tc_in_body(x_ref, wfc_ref, bfc_ref, wnbr_ref, wroot_ref, b_ref, g_ref, r_ref):
    h = jnp.dot(x_ref[...], wfc_ref[...], preferred_element_type=jnp.float32)
    h = h + bfc_ref[...]
    g_ref[...] = _mask_rows(jnp.dot(h, wnbr_ref[...], preferred_element_type=jnp.float32))
    r_ref[...] = jnp.dot(h, wroot_ref[...], preferred_element_type=jnp.float32) + b_ref[...]


def _tc_mid_body(r_ref_in, parts_ref, wnbr_ref, wroot_ref, b_ref, g_ref, r_ref):
    h = jnp.tanh(r_ref_in[...] + parts_ref[0, :, :D_H] + parts_ref[1, :, :D_H])
    g_ref[...] = _mask_rows(jnp.dot(h, wnbr_ref[...], preferred_element_type=jnp.float32))
    r_ref[...] = jnp.dot(h, wroot_ref[...], preferred_element_type=jnp.float32) + b_ref[...]


def _tc_out_body(r_ref_in, parts_ref, wcat_ref, bcat_ref, out_ref):
    h = jnp.tanh(r_ref_in[...] + parts_ref[0, :, :D_H] + parts_ref[1, :, :D_H])
    out_ref[...] = jnp.dot(h, wcat_ref[...], preferred_element_type=jnp.float32) + bcat_ref[...]


def _sc_conv_body(g_hbm, src_hbm, dst_hbm, zeros_hbm, out_hbm,
                  src_v, dst_v, rows_v, agg_sh, sem):
    c = lax.axis_index("c")
    s = lax.axis_index("s")
    wid = s * NC + c

    @pl.when(s == 0)
    def _():
        pltpu.sync_copy(zeros_hbm, agg_sh)

    plsc.subcore_barrier()

    def body(j, carry):
        base = (wid * CPW + j) * CH
        pltpu.sync_copy(src_hbm.at[pl.ds(base, CH)], src_v)
        pltpu.sync_copy(dst_hbm.at[pl.ds(base, CH)], dst_v)
        pltpu.async_copy(g_hbm.at[src_v], rows_v, sem).wait()
        pltpu.sync_copy(rows_v, agg_sh.at[dst_v], add=True)
        return carry

    lax.fori_loop(0, CPW, body, 0)
    plsc.subcore_barrier()
    pltpu.sync_copy(agg_sh.at[pl.ds(s * RPT, RPT)],
                    out_hbm.at[pl.ds(c * N_PAD + s * RPT, RPT)])


def _sc_conv(g, src, dst, zeros):
    f = pl.kernel(
        _sc_conv_body,
        out_type=jax.ShapeDtypeStruct((NC * N_PAD, GW), jnp.float32),
        mesh=plsc.VectorSubcoreMesh(core_axis_name="c", subcore_axis_name="s"),
        scratch_types=[
            pltpu.VMEM((CH,), jnp.int32),
            pltpu.VMEM((CH,), jnp.int32),
            pltpu.VMEM((CH, GW), jnp.float32),
            pltpu.VMEM_SHARED((N_PAD, GW), jnp.float32),
            pltpu.SemaphoreType.DMA,
        ],
    )
    return f(g, src, dst, zeros).reshape(NC, N_PAD, GW)


def kernel(x, edge_index, W_fc, b_fc, W_root1, W_nbr1, b1,
           W_root2, W_nbr2, b2, W_logit, b_logit, W_val, b_val):
    x_pad = jnp.concatenate([x, jnp.zeros((N_PAD - N, D_IN), x.dtype)], axis=0)
    fill = jnp.full((E_PAD - E,), N, dtype=jnp.int32)
    src = jnp.concatenate([edge_index[0], fill])
    dst = jnp.concatenate([edge_index[1], fill])
    zeros = jnp.zeros((N_PAD, GW), jnp.float32)

    W_cat = jnp.concatenate([W_logit, W_val], axis=1)
    b_cat = jnp.concatenate([b_logit, b_val]).reshape(1, D_OUT)
    W_nbr1p = jnp.pad(W_nbr1, ((0, 0), (0, GW - D_H)))
    W_nbr2p = jnp.pad(W_nbr2, ((0, 0), (0, GW - D_H)))

    mk2 = functools.partial(
        pl.pallas_call,
        out_shape=[jax.ShapeDtypeStruct((N_PAD, GW), jnp.float32),
                   jax.ShapeDtypeStruct((N_PAD, D_H), jnp.float32)],
    )
    g1, r1 = mk2(_tc_in_body)(x_pad, W_fc, b_fc.reshape(1, D_H),
                              W_nbr1p, W_root1, b1.reshape(1, D_H))
    parts1 = _sc_conv(g1, src, dst, zeros)
    g2, r2 = mk2(_tc_mid_body)(r1, parts1, W_nbr2p, W_root2, b2.reshape(1, D_H))
    parts2 = _sc_conv(g2, src, dst, zeros)
    out = pl.pallas_call(
        _tc_out_body,
        out_shape=jax.ShapeDtypeStruct((N_PAD, D_OUT), jnp.float32),
    )(r2, parts2, W_cat, b_cat)
    return out[:N]
